# baseline (speedup 1.0000x reference)
import jax
import jax.numpy as jnp
from jax.experimental import pallas as pl
from jax.experimental.pallas import tpu as pltpu

B, R, L, IN, OUT = 128, 6, 128, 128, 64
NB = 32
SUB = 72
KA = 136


def _aergcn_body(text_ref, adj_ref, waug_ref, out_ref):
    onescol = jnp.concatenate(
        [jnp.ones((L, 1), jnp.bfloat16),
         jnp.zeros((L, KA - IN - 1), jnp.bfloat16)], axis=1)
    for b in range(NB):
        x = jnp.concatenate(
            [text_ref[b].astype(jnp.bfloat16), onescol], axis=1)
        accT = jnp.zeros((OUT, L), jnp.float32)
        zsum = jnp.zeros((1, L), jnp.float32)
        for r in range(R):
            hT = jax.lax.dot_general(
                waug_ref[r], x, (((1,), (1,)), ((), ())),
                preferred_element_type=jnp.float32).astype(jnp.bfloat16)
            a = adj_ref[b, r].astype(jnp.bfloat16)
            mdT = jax.lax.dot_general(
                hT, a, (((1,), (1,)), ((), ())),
                preferred_element_type=jnp.float32)
            msgT = mdT[:OUT]
            snum = mdT[OUT:OUT + 1]
            den = mdT[OUT + 1:OUT + 2]
            recip = jnp.where(den == 0.0, 1.0, 1.0 / den)
            e = jnp.exp(snum * recip)
            zsum = zsum + e
            accT = accT + (e * recip) * msgT
        outT = accT / zsum
        out_ref[b] = outT.T


@jax.jit
def kernel(text, adj, weight, score_w, score_b):
    wT = weight.transpose(0, 2, 1)
    svT = jnp.einsum('rfo,o->rf', weight, score_w[0])[:, None, :]
    rows = jnp.concatenate(
        [wT, svT, jnp.zeros((R, SUB - OUT - 1, IN), jnp.float32)], axis=1)
    waug = jnp.concatenate(
        [rows, jnp.zeros((R, SUB, KA - IN), jnp.float32)], axis=2)
    waug = waug.at[:, OUT + 1, IN].set(1.0).astype(jnp.bfloat16)
    del score_b
    grid = (B // NB,)
    return pl.pallas_call(
        _aergcn_body,
        grid=grid,
        in_specs=[
            pl.BlockSpec((NB, L, IN), lambda i: (i, 0, 0)),
            pl.BlockSpec((NB, R, L, L), lambda i: (i, 0, 0, 0)),
            pl.BlockSpec((R, SUB, KA), lambda i: (0, 0, 0)),
        ],
        out_specs=pl.BlockSpec((NB, L, OUT), lambda i: (i, 0, 0)),
        out_shape=jax.ShapeDtypeStruct((B, L, OUT), jnp.float32),
    )(text, adj, waug)


# speedup vs baseline: 1.0193x; 1.0193x over previous
import jax
import jax.numpy as jnp
from jax.experimental import pallas as pl
from jax.experimental.pallas import tpu as pltpu

B, R, L, IN, OUT = 128, 6, 128, 128, 64
NB = 16
SUB = 72


def _aergcn_body(text_ref, adj_ref, waug_ref, out_ref):
    onesrow = jax.lax.broadcasted_iota(jnp.int32, (SUB, L), 0) == OUT + 1
    for b in range(NB):
        x = text_ref[b].astype(jnp.bfloat16)
        accT = jnp.zeros((OUT, L), jnp.float32)
        zsum = jnp.zeros((1, L), jnp.float32)
        for r in range(R):
            hT = jax.lax.dot_general(
                waug_ref[r], x, (((1,), (1,)), ((), ())),
                preferred_element_type=jnp.float32).astype(jnp.bfloat16)
            hT = jnp.where(onesrow, jnp.bfloat16(1.0), hT)
            a = adj_ref[b, r].astype(jnp.bfloat16)
            mdT = jax.lax.dot_general(
                hT, a, (((1,), (1,)), ((), ())),
                preferred_element_type=jnp.float32)
            msgT = mdT[:OUT]
            snum = mdT[OUT:OUT + 1]
            den = mdT[OUT + 1:OUT + 2]
            recip = jnp.where(den == 0.0, 1.0, 1.0 / den)
            e = jnp.exp(snum * recip)
            zsum = zsum + e
            accT = accT + (e * recip) * msgT
        outT = accT / zsum
        out_ref[b] = outT.T


@jax.jit
def kernel(text, adj, weight, score_w, score_b):
    wT = weight.transpose(0, 2, 1)
    svT = jnp.einsum('rfo,o->rf', weight, score_w[0])[:, None, :]
    waug = jnp.concatenate(
        [wT, svT, jnp.zeros((R, SUB - OUT - 1, IN), jnp.float32)],
        axis=1).astype(jnp.bfloat16)
    del score_b
    grid = (B // NB,)
    return pl.pallas_call(
        _aergcn_body,
        grid=grid,
        in_specs=[
            pl.BlockSpec((NB, L, IN), lambda i: (i, 0, 0)),
            pl.BlockSpec((NB, R, L, L), lambda i: (i, 0, 0, 0)),
            pl.BlockSpec((R, SUB, IN), lambda i: (0, 0, 0)),
        ],
        out_specs=pl.BlockSpec((NB, L, OUT), lambda i: (i, 0, 0)),
        out_shape=jax.ShapeDtypeStruct((B, L, OUT), jnp.float32),
    )(text, adj, waug)


# speedup vs baseline: 1.0228x; 1.0034x over previous
import jax
import jax.numpy as jnp
from jax.experimental import pallas as pl

B, R, L, IN, OUT = 128, 6, 128, 128, 64
NB = 16
SUB = 72


def _aergcn_body(text_ref, adj_ref, waug_ref, out_ref):
    onesrow = jax.lax.broadcasted_iota(jnp.int32, (SUB, L), 0) == OUT + 1
    for b in range(NB):
        x = text_ref[b].astype(jnp.bfloat16)
        accT = jnp.zeros((OUT, L), jnp.float32)
        zsum = jnp.zeros((1, L), jnp.float32)
        for r in range(R):
            hT = jax.lax.dot_general(
                waug_ref[r], x, (((1,), (1,)), ((), ())),
                preferred_element_type=jnp.float32).astype(jnp.bfloat16)
            hT = jnp.where(onesrow, jnp.bfloat16(1.0), hT)
            a = adj_ref[b, r].astype(jnp.bfloat16)
            mdT = jax.lax.dot_general(
                hT, a, (((1,), (1,)), ((), ())),
                preferred_element_type=jnp.float32)
            msgT = mdT[:OUT]
            snum = mdT[OUT:OUT + 1]
            den = mdT[OUT + 1:OUT + 2]
            recip = jnp.where(den == 0.0, 1.0, 1.0 / den)
            e = jnp.exp(snum * recip)
            zsum = zsum + e
            accT = accT + (e * recip) * msgT
        outT = accT / zsum
        out_ref[b] = outT.T


@jax.jit
def kernel(text, adj, weight, score_w, score_b):
    wT = weight.transpose(0, 2, 1)
    svT = jnp.einsum('rfo,o->rf', weight, score_w[0])[:, None, :]
    waug = jnp.concatenate(
        [wT, svT, jnp.zeros((R, SUB - OUT - 1, IN), jnp.float32)],
        axis=1).astype(jnp.bfloat16)
    del score_b
    grid = (B // NB,)
    return pl.pallas_call(
        _aergcn_body,
        grid=grid,
        in_specs=[
            pl.BlockSpec((NB, L, IN), lambda i: (i, 0, 0)),
            pl.BlockSpec((NB, R, L, L), lambda i: (i, 0, 0, 0)),
            pl.BlockSpec((R, SUB, IN), lambda i: (0, 0, 0)),
        ],
        out_specs=pl.BlockSpec((NB, L, OUT), lambda i: (i, 0, 0)),
        out_shape=jax.ShapeDtypeStruct((B, L, OUT), jnp.float32),
    )(text, adj, waug)
